# baseline (speedup 1.0000x reference)
import functools

import jax
import jax.numpy as jnp
from jax import lax
from jax.experimental import pallas as pl
from jax.experimental.pallas import tpu as pltpu
from jax.experimental.pallas import tpu_sc as plsc

NC = 2
NS = 16
NW = NC * NS
C = 128


def _rsqrt16(d):
    i = lax.bitcast_convert_type(d, jnp.int32)
    i = jnp.int32(0x5F3759DF) - jnp.right_shift(i, 1)
    y = lax.bitcast_convert_type(i, jnp.float32)
    hd = 0.5 * d
    for _ in range(3):
        y = y * (1.5 - hd * y * y)
    return y


def _make_deg_kernel(npad, chunks, seg):
    mesh = plsc.VectorSubcoreMesh(core_axis_name="c", subcore_axis_name="s")

    @functools.partial(
        pl.kernel,
        out_type=jax.ShapeDtypeStruct((NC, npad), jnp.float32),
        mesh=mesh,
        scratch_types=[
            pltpu.VMEM((chunks, C), jnp.int32),
            pltpu.VMEM((chunks, C), jnp.float32),
            pltpu.VMEM((seg,), jnp.float32),
            pltpu.VMEM_SHARED((npad,), jnp.float32),
        ],
        compiler_params=pltpu.CompilerParams(needs_layout_passes=False, use_tc_tiling_on_sc=False),
    )
    def deg_kernel(col_hbm, ew_hbm, deg_out, col_v, ew_v, zero_v, deg_sh):
        cid = lax.axis_index("c")
        sid = lax.axis_index("s")
        wid = cid * NS + sid
        pltpu.sync_copy(col_hbm.at[wid], col_v)
        pltpu.sync_copy(ew_hbm.at[wid], ew_v)

        def zbody(i, carry):
            zero_v[pl.ds(i * 16, 16)] = jnp.zeros((16,), jnp.float32)
            return carry

        lax.fori_loop(0, seg // 16, zbody, 0)
        pltpu.sync_copy(zero_v, deg_sh.at[pl.ds(sid * seg, seg)])
        plsc.subcore_barrier()

        def body(j, carry):
            pltpu.sync_copy(ew_v.at[j], deg_sh.at[col_v.at[j]], add=True)
            return carry

        lax.fori_loop(0, chunks, body, 0)
        plsc.subcore_barrier()
        pltpu.sync_copy(deg_sh.at[pl.ds(sid * seg, seg)],
                        deg_out.at[cid, pl.ds(sid * seg, seg)])

    return deg_kernel


def _make_gcn_kernel(npad, chunks, seg, d_out):
    mesh = plsc.VectorSubcoreMesh(core_axis_name="c", subcore_axis_name="s")

    @functools.partial(
        pl.kernel,
        out_type=jax.ShapeDtypeStruct((NC, npad, d_out), jnp.float32),
        mesh=mesh,
        scratch_types=[
            pltpu.VMEM((2, 4, 3, C), jnp.int32),
            pltpu.VMEM((NC, seg), jnp.float32),
            pltpu.VMEM((npad,), jnp.float32),
            pltpu.VMEM((2, C, d_out), jnp.float32),
            pltpu.VMEM((C,), jnp.float32),
            pltpu.VMEM_SHARED((npad, d_out), jnp.float32),
            pltpu.SemaphoreType.DMA,
            pltpu.SemaphoreType.DMA,
            pltpu.SemaphoreType.DMA,
            pltpu.SemaphoreType.DMA,
            pltpu.SemaphoreType.DMA,
            pltpu.SemaphoreType.DMA,
        ],
        compiler_params=pltpu.CompilerParams(
            needs_layout_passes=False, use_tc_tiling_on_sc=False),
    )
    def gcn_kernel(rc_hbm, h_hbm, deg_hbm, s_out,
                   rc_v, degb_v, dinv_v, rows_v, coef_v, acc_sh,
                   gsem0, gsem1, ssem0, ssem1, rsem0, rsem1):
        cid = lax.axis_index("c")
        sid = lax.axis_index("s")
        wid = cid * NS + sid
        gsems = (gsem0, gsem1)
        ssems = (ssem0, ssem1)
        rsems = (rsem0, rsem1)

        def dblk(bb, carry):
            pltpu.sync_copy(deg_hbm.at[:, pl.ds(bb * seg, seg)], degb_v)

            def dbody(i, carry2):
                sl = pl.ds(i * 16, 16)
                d = degb_v[0, sl] + degb_v[1, sl]
                dinv_v[pl.ds(bb * seg + i * 16, 16)] = _rsqrt16(d)
                return carry2

            lax.fori_loop(0, seg // 16, dbody, 0)
            return carry

        lax.fori_loop(0, npad // seg, dblk, 0)

        def zbody(r, carry):
            for cb in range(d_out // 16):
                rows_v[0, r, pl.ds(cb * 16, 16)] = jnp.zeros((16,), jnp.float32)
            return carry

        lax.fori_loop(0, C, zbody, 0)
        for t in range(seg // C):
            pltpu.sync_copy(rows_v.at[0], acc_sh.at[pl.ds(sid * seg + t * C, C)])

        nblk = chunks // 4
        pltpu.sync_copy(rc_hbm.at[wid, pl.ds(0, 4)], rc_v.at[0])
        if nblk > 1:
            pltpu.async_copy(rc_hbm.at[wid, pl.ds(4, 4)], rc_v.at[1], rsem1)
        pltpu.async_copy(h_hbm.at[rc_v.at[0, 0, 0]], rows_v.at[0], gsem0)
        plsc.subcore_barrier()

        def step(j, p):
            q = 1 - p
            blk = jnp.bitwise_and(jnp.right_shift(j, 2), 1)
            cpos = jnp.bitwise_and(j, 3)
            blk1 = jnp.bitwise_and(jnp.right_shift(j + 1, 2), 1)
            cpos1 = jnp.bitwise_and(j + 1, 3)

            @pl.when(j >= 1)
            def _():
                pltpu.make_async_copy(
                    rows_v.at[q], acc_sh.at[rc_v.at[0, 0, 1]], ssems[q]).wait()

            @pl.when(jnp.logical_and(cpos == 3, j + 1 < chunks))
            def _():
                @pl.when(blk1 == 0)
                def _():
                    pltpu.make_async_copy(
                        rc_hbm.at[wid, pl.ds(0, 4)], rc_v.at[0], rsem0).wait()

                @pl.when(blk1 == 1)
                def _():
                    pltpu.make_async_copy(
                        rc_hbm.at[wid, pl.ds(0, 4)], rc_v.at[1], rsem1).wait()

            @pl.when(j + 1 < chunks)
            def _():
                pltpu.async_copy(
                    h_hbm.at[rc_v.at[blk1, cpos1, 0]], rows_v.at[q], gsems[q])

            @pl.when(jnp.logical_and(
                cpos == 1, jnp.logical_and(j > 1, j + 3 < chunks)))
            def _():
                nb = jnp.right_shift(j + 3, 2)

                @pl.when(jnp.bitwise_and(nb, 1) == 0)
                def _():
                    pltpu.async_copy(rc_hbm.at[wid, pl.ds(nb * 4, 4)],
                                     rc_v.at[0], rsem0)

                @pl.when(jnp.bitwise_and(nb, 1) == 1)
                def _():
                    pltpu.async_copy(rc_hbm.at[wid, pl.ds(nb * 4, 4)],
                                     rc_v.at[1], rsem1)

            pltpu.make_async_copy(
                h_hbm.at[rc_v.at[0, 0, 0]], rows_v.at[p], gsems[p]).wait()

            for k in range(C // 16):
                sl = pl.ds(k * 16, 16)
                r16 = rc_v[blk, cpos, 0, sl]
                c16 = rc_v[blk, cpos, 1, sl]
                w16 = lax.bitcast_convert_type(rc_v[blk, cpos, 2, sl],
                                               jnp.float32)
                dr = plsc.load_gather(dinv_v, [r16])
                dc = plsc.load_gather(dinv_v, [c16])
                coef_v[sl] = w16 * dr * dc

            def scale(g, carry2):
                c16 = coef_v[pl.ds(g * 16, 16)]
                for l in range(16):
                    s = c16[l]
                    r = g * 16 + l
                    for cb in range(d_out // 16):
                        sl2 = pl.ds(cb * 16, 16)
                        rows_v[p, r, sl2] = rows_v[p, r, sl2] * s
                return carry2

            lax.fori_loop(0, C // 16, scale, 0)
            pltpu.async_copy(
                rows_v.at[p], acc_sh.at[rc_v.at[blk, cpos, 1]], ssems[p],
                add=True)

        def pair(t, carry):
            step(2 * t, 0)
            step(2 * t + 1, 1)
            return carry

        lax.fori_loop(0, chunks // 2, pair, 0)
        pltpu.make_async_copy(
            rows_v.at[1], acc_sh.at[rc_v.at[0, 0, 1]], ssems[1]).wait()
        plsc.subcore_barrier()
        pltpu.sync_copy(acc_sh.at[pl.ds(sid * seg, seg)],
                        s_out.at[cid, pl.ds(sid * seg, seg)])

    return gcn_kernel


def _matmul(x, W):
    n, d_in = x.shape
    d_out = W.shape[1]
    bs = 1000 if n % 1000 == 0 else n

    def body(x_ref, w_ref, o_ref):
        o_ref[...] = jnp.dot(x_ref[...], w_ref[...],
                             preferred_element_type=jnp.float32)

    return pl.pallas_call(
        body,
        grid=(n // bs,),
        in_specs=[
            pl.BlockSpec((bs, d_in), lambda i: (i, 0)),
            pl.BlockSpec((d_in, d_out), lambda i: (0, 0)),
        ],
        out_specs=pl.BlockSpec((bs, d_out), lambda i: (i, 0)),
        out_shape=jax.ShapeDtypeStruct((n, d_out), jnp.float32),
    )(x, W)


def _combine(S, b2):
    _, n, d_out = S.shape
    bs = 1000 if n % 1000 == 0 else n

    def body(s_ref, b_ref, o_ref):
        o_ref[...] = s_ref[0] + s_ref[1] + b_ref[...]

    return pl.pallas_call(
        body,
        grid=(n // bs,),
        in_specs=[
            pl.BlockSpec((NC, bs, d_out), lambda i: (0, i, 0)),
            pl.BlockSpec((1, d_out), lambda i: (0, 0)),
        ],
        out_specs=pl.BlockSpec((bs, d_out), lambda i: (i, 0)),
        out_shape=jax.ShapeDtypeStruct((n, d_out), jnp.float32),
    )(S, b2)


def kernel(x, edge_index, edge_weight, W, b):
    n = x.shape[0]
    e = edge_index.shape[1]
    d_out = W.shape[1]

    row = edge_index[0].astype(jnp.int32)
    col = edge_index[1].astype(jnp.int32)
    loop = jnp.arange(n, dtype=jnp.int32)
    rows = jnp.concatenate([row, loop])
    cols = jnp.concatenate([col, loop])
    ews = jnp.concatenate([edge_weight.astype(jnp.float32),
                           jnp.ones((n,), jnp.float32)])

    per = NW * C
    chunks = -(-(e + n) // per)
    chunks = ((chunks + 3) // 4) * 4
    epad = per * chunks
    pad = epad - (e + n)
    spread = jnp.arange(pad, dtype=jnp.int32) % jnp.int32(n)
    rows_p = jnp.concatenate([rows, spread])
    cols_p = jnp.concatenate([cols, spread])
    ews_p = jnp.concatenate([ews, jnp.zeros((pad,), jnp.float32)])

    seg = ((n + NS * C - 1) // (NS * C)) * C
    npad = NS * seg
    dh = d_out // NC

    deg = _make_deg_kernel(npad, chunks, seg)(
        cols_p.reshape(NW, chunks, C), ews_p.reshape(NW, chunks, C))
    h = _matmul(x, W)
    ew_bits = lax.bitcast_convert_type(ews_p, jnp.int32)
    rc = jnp.stack([rows_p.reshape(NW, chunks, C),
                    cols_p.reshape(NW, chunks, C),
                    ew_bits.reshape(NW, chunks, C)], axis=2)
    S = _make_gcn_kernel(npad, chunks, seg, d_out)(rc, h, deg)
    out = _combine(S[:, :n, :], b.reshape(1, d_out).astype(jnp.float32))
    return out


# speedup vs baseline: 1.0143x; 1.0143x over previous
import functools

import jax
import jax.numpy as jnp
from jax import lax
from jax.experimental import pallas as pl
from jax.experimental.pallas import tpu as pltpu
from jax.experimental.pallas import tpu_sc as plsc

NC = 2
NS = 16
NW = NC * NS
C = 128


def _rsqrt16(d):
    i = lax.bitcast_convert_type(d, jnp.int32)
    i = jnp.int32(0x5F3759DF) - jnp.right_shift(i, 1)
    y = lax.bitcast_convert_type(i, jnp.float32)
    hd = 0.5 * d
    for _ in range(3):
        y = y * (1.5 - hd * y * y)
    return y


def _make_deg_kernel(npad, chunks, seg):
    mesh = plsc.VectorSubcoreMesh(core_axis_name="c", subcore_axis_name="s")

    @functools.partial(
        pl.kernel,
        out_type=jax.ShapeDtypeStruct((NC, npad), jnp.float32),
        mesh=mesh,
        scratch_types=[
            pltpu.VMEM((chunks, C), jnp.int32),
            pltpu.VMEM((chunks, C), jnp.float32),
            pltpu.VMEM((seg,), jnp.float32),
            pltpu.VMEM_SHARED((npad,), jnp.float32),
        ],
        compiler_params=pltpu.CompilerParams(needs_layout_passes=False, use_tc_tiling_on_sc=False),
    )
    def deg_kernel(col_hbm, ew_hbm, deg_out, col_v, ew_v, zero_v, deg_sh):
        cid = lax.axis_index("c")
        sid = lax.axis_index("s")
        wid = cid * NS + sid
        pltpu.sync_copy(col_hbm.at[wid], col_v)
        pltpu.sync_copy(ew_hbm.at[wid], ew_v)

        def zbody(i, carry):
            zero_v[pl.ds(i * 16, 16)] = jnp.zeros((16,), jnp.float32)
            return carry

        lax.fori_loop(0, seg // 16, zbody, 0)
        pltpu.sync_copy(zero_v, deg_sh.at[pl.ds(sid * seg, seg)])
        plsc.subcore_barrier()

        def body(j, carry):
            pltpu.sync_copy(ew_v.at[j], deg_sh.at[col_v.at[j]], add=True)
            return carry

        lax.fori_loop(0, chunks, body, 0)
        plsc.subcore_barrier()
        pltpu.sync_copy(deg_sh.at[pl.ds(sid * seg, seg)],
                        deg_out.at[cid, pl.ds(sid * seg, seg)])

    return deg_kernel


def _make_gcn_kernel(npad, chunks, seg, d_out):
    mesh = plsc.VectorSubcoreMesh(core_axis_name="c", subcore_axis_name="s")

    @functools.partial(
        pl.kernel,
        out_type=jax.ShapeDtypeStruct((NC, npad, d_out), jnp.float32),
        mesh=mesh,
        scratch_types=[
            pltpu.VMEM((2, 4, 3, C), jnp.int32),
            pltpu.VMEM((NC, seg), jnp.float32),
            pltpu.VMEM((npad,), jnp.float32),
            pltpu.VMEM((2, C, d_out), jnp.float32),
            pltpu.VMEM((C,), jnp.float32),
            pltpu.VMEM_SHARED((npad, d_out), jnp.float32),
            pltpu.SemaphoreType.DMA,
            pltpu.SemaphoreType.DMA,
            pltpu.SemaphoreType.DMA,
            pltpu.SemaphoreType.DMA,
            pltpu.SemaphoreType.DMA,
            pltpu.SemaphoreType.DMA,
        ],
        compiler_params=pltpu.CompilerParams(
            needs_layout_passes=False, use_tc_tiling_on_sc=False),
    )
    def gcn_kernel(rc_hbm, h_hbm, deg_hbm, s_out,
                   rc_v, degb_v, dinv_v, rows_v, coef_v, acc_sh,
                   gsem0, gsem1, ssem0, ssem1, rsem0, rsem1):
        cid = lax.axis_index("c")
        sid = lax.axis_index("s")
        wid = cid * NS + sid
        gsems = (gsem0, gsem1)
        ssems = (ssem0, ssem1)
        rsems = (rsem0, rsem1)

        def dblk(bb, carry):
            pltpu.sync_copy(deg_hbm.at[:, pl.ds(bb * seg, seg)], degb_v)

            def dbody(i, carry2):
                sl = pl.ds(i * 16, 16)
                d = degb_v[0, sl] + degb_v[1, sl]
                dinv_v[pl.ds(bb * seg + i * 16, 16)] = _rsqrt16(d)
                return carry2

            lax.fori_loop(0, seg // 16, dbody, 0)
            return carry

        lax.fori_loop(0, npad // seg, dblk, 0)

        def zbody(r, carry):
            for cb in range(d_out // 16):
                rows_v[0, r, pl.ds(cb * 16, 16)] = jnp.zeros((16,), jnp.float32)
            return carry

        lax.fori_loop(0, C, zbody, 0)
        for t in range(seg // C):
            pltpu.sync_copy(rows_v.at[0], acc_sh.at[pl.ds(sid * seg + t * C, C)])

        nblk = chunks // 4
        pltpu.sync_copy(rc_hbm.at[wid, pl.ds(0, 4)], rc_v.at[0])
        if nblk > 1:
            pltpu.async_copy(rc_hbm.at[wid, pl.ds(4, 4)], rc_v.at[1], rsem1)
        pltpu.async_copy(h_hbm.at[rc_v.at[0, 0, 0]], rows_v.at[0], gsem0)
        plsc.subcore_barrier()

        def step(j, p):
            q = 1 - p
            blk = jnp.bitwise_and(jnp.right_shift(j, 2), 1)
            cpos = jnp.bitwise_and(j, 3)
            blk1 = jnp.bitwise_and(jnp.right_shift(j + 1, 2), 1)
            cpos1 = jnp.bitwise_and(j + 1, 3)

            @pl.when(j >= 1)
            def _():
                pltpu.make_async_copy(
                    rows_v.at[q], acc_sh.at[rc_v.at[0, 0, 1]], ssems[q]).wait()

            @pl.when(jnp.logical_and(cpos == 3, j + 1 < chunks))
            def _():
                @pl.when(blk1 == 0)
                def _():
                    pltpu.make_async_copy(
                        rc_hbm.at[wid, pl.ds(0, 4)], rc_v.at[0], rsem0).wait()

                @pl.when(blk1 == 1)
                def _():
                    pltpu.make_async_copy(
                        rc_hbm.at[wid, pl.ds(0, 4)], rc_v.at[1], rsem1).wait()

            @pl.when(j + 1 < chunks)
            def _():
                pltpu.async_copy(
                    h_hbm.at[rc_v.at[blk1, cpos1, 0]], rows_v.at[q], gsems[q])

            @pl.when(jnp.logical_and(
                cpos == 1, jnp.logical_and(j > 1, j + 3 < chunks)))
            def _():
                nb = jnp.right_shift(j + 3, 2)

                @pl.when(jnp.bitwise_and(nb, 1) == 0)
                def _():
                    pltpu.async_copy(rc_hbm.at[wid, pl.ds(nb * 4, 4)],
                                     rc_v.at[0], rsem0)

                @pl.when(jnp.bitwise_and(nb, 1) == 1)
                def _():
                    pltpu.async_copy(rc_hbm.at[wid, pl.ds(nb * 4, 4)],
                                     rc_v.at[1], rsem1)

            for k in range(C // 16):
                sl = pl.ds(k * 16, 16)
                r16 = rc_v[blk, cpos, 0, sl]
                c16 = rc_v[blk, cpos, 1, sl]
                w16 = lax.bitcast_convert_type(rc_v[blk, cpos, 2, sl],
                                               jnp.float32)
                dr = plsc.load_gather(dinv_v, [r16])
                dc = plsc.load_gather(dinv_v, [c16])
                coef_v[sl] = w16 * dr * dc

            pltpu.make_async_copy(
                h_hbm.at[rc_v.at[0, 0, 0]], rows_v.at[p], gsems[p]).wait()

            def scale(g, carry2):
                c16 = coef_v[pl.ds(g * 16, 16)]
                for l in range(16):
                    s = c16[l]
                    r = g * 16 + l
                    for cb in range(d_out // 16):
                        sl2 = pl.ds(cb * 16, 16)
                        rows_v[p, r, sl2] = rows_v[p, r, sl2] * s
                return carry2

            lax.fori_loop(0, C // 16, scale, 0, unroll=2)
            pltpu.async_copy(
                rows_v.at[p], acc_sh.at[rc_v.at[blk, cpos, 1]], ssems[p],
                add=True)

        def pair(t, carry):
            step(2 * t, 0)
            step(2 * t + 1, 1)
            return carry

        lax.fori_loop(0, chunks // 2, pair, 0)
        pltpu.make_async_copy(
            rows_v.at[1], acc_sh.at[rc_v.at[0, 0, 1]], ssems[1]).wait()
        plsc.subcore_barrier()
        pltpu.sync_copy(acc_sh.at[pl.ds(sid * seg, seg)],
                        s_out.at[cid, pl.ds(sid * seg, seg)])

    return gcn_kernel


def _matmul(x, W):
    n, d_in = x.shape
    d_out = W.shape[1]
    bs = 1000 if n % 1000 == 0 else n

    def body(x_ref, w_ref, o_ref):
        o_ref[...] = jnp.dot(x_ref[...], w_ref[...],
                             preferred_element_type=jnp.float32)

    return pl.pallas_call(
        body,
        grid=(n // bs,),
        in_specs=[
            pl.BlockSpec((bs, d_in), lambda i: (i, 0)),
            pl.BlockSpec((d_in, d_out), lambda i: (0, 0)),
        ],
        out_specs=pl.BlockSpec((bs, d_out), lambda i: (i, 0)),
        out_shape=jax.ShapeDtypeStruct((n, d_out), jnp.float32),
    )(x, W)


def _combine(S, b2):
    _, n, d_out = S.shape
    bs = 1000 if n % 1000 == 0 else n

    def body(s_ref, b_ref, o_ref):
        o_ref[...] = s_ref[0] + s_ref[1] + b_ref[...]

    return pl.pallas_call(
        body,
        grid=(n // bs,),
        in_specs=[
            pl.BlockSpec((NC, bs, d_out), lambda i: (0, i, 0)),
            pl.BlockSpec((1, d_out), lambda i: (0, 0)),
        ],
        out_specs=pl.BlockSpec((bs, d_out), lambda i: (i, 0)),
        out_shape=jax.ShapeDtypeStruct((n, d_out), jnp.float32),
    )(S, b2)


def kernel(x, edge_index, edge_weight, W, b):
    n = x.shape[0]
    e = edge_index.shape[1]
    d_out = W.shape[1]

    row = edge_index[0].astype(jnp.int32)
    col = edge_index[1].astype(jnp.int32)
    loop = jnp.arange(n, dtype=jnp.int32)
    rows = jnp.concatenate([row, loop])
    cols = jnp.concatenate([col, loop])
    ews = jnp.concatenate([edge_weight.astype(jnp.float32),
                           jnp.ones((n,), jnp.float32)])

    per = NW * C
    chunks = -(-(e + n) // per)
    chunks = ((chunks + 3) // 4) * 4
    epad = per * chunks
    pad = epad - (e + n)
    spread = jnp.arange(pad, dtype=jnp.int32) % jnp.int32(n)
    rows_p = jnp.concatenate([rows, spread])
    cols_p = jnp.concatenate([cols, spread])
    ews_p = jnp.concatenate([ews, jnp.zeros((pad,), jnp.float32)])

    seg = ((n + NS * C - 1) // (NS * C)) * C
    npad = NS * seg
    dh = d_out // NC

    deg = _make_deg_kernel(npad, chunks, seg)(
        cols_p.reshape(NW, chunks, C), ews_p.reshape(NW, chunks, C))
    h = _matmul(x, W)
    ew_bits = lax.bitcast_convert_type(ews_p, jnp.int32)
    rc = jnp.stack([rows_p.reshape(NW, chunks, C),
                    cols_p.reshape(NW, chunks, C),
                    ew_bits.reshape(NW, chunks, C)], axis=2)
    S = _make_gcn_kernel(npad, chunks, seg, d_out)(rc, h, deg)
    out = _combine(S[:, :n, :], b.reshape(1, d_out).astype(jnp.float32))
    return out
